# baseline (speedup 1.0000x reference)
import functools

import jax
import jax.numpy as jnp
from jax import lax
from jax.experimental import pallas as pl
from jax.experimental.pallas import tpu as pltpu
from jax.experimental.pallas import tpu_sc as plsc

_N = 4096
_E = 128
_K = 26
_NORM = 9.0
_NR = 25.0

_NW = 32
_TPW = _N // _NW
_TCH = 4
_RCH = _TCH * _K
_NCH = _TPW // _TCH
_RPW = _TPW * _K
_RPAD = 112
_NFLAT = _N * _K


def _sc_body(w_hbm, inp_hbm, idx_hbm, b_hbm, nz_hbm,
             dot_out, b_out, n_out,
             in_v, idx_v, rows0, rows1, acc_v,
             out0, out1, bv0, bv1, nv0, nv1,
             sem0, sem1, semb0, semb1, semn0, semn1, semw0, semw1):
    nc = 2
    wid = lax.axis_index("s") * nc + lax.axis_index("c")
    tok0 = wid * _TPW

    rows_b = (rows0, rows1)
    out_b = (out0, out1)
    bv_b = (bv0, bv1)
    nv_b = (nv0, nv1)
    semr = (sem0, sem1)
    semb = (semb0, semb1)
    semn = (semn0, semn1)
    semw = (semw0, semw1)

    pltpu.sync_copy(idx_hbm.at[pl.ds(wid * _NCH, _NCH)], idx_v)
    pltpu.sync_copy(inp_hbm.at[pl.ds(tok0, _TPW)], in_v)

    def rows_copy(cc, buf):
        return pltpu.make_async_copy(
            w_hbm.at[idx_v.at[cc]], rows_b[buf], semr[buf])

    def bn_copies(cc, buf):
        return (
            pltpu.make_async_copy(b_hbm.at[idx_v.at[cc]], bv_b[buf],
                                  semb[buf]),
            pltpu.make_async_copy(nz_hbm.at[idx_v.at[cc]], nv_b[buf],
                                  semn[buf]),
        )

    def out_copies(cc, buf):
        base = wid * _RPW + cc * _RCH
        return (
            pltpu.make_async_copy(out_b[buf].at[pl.ds(0, _RCH)],
                                  dot_out.at[pl.ds(base, _RCH)], semw[buf]),
            pltpu.make_async_copy(bv_b[buf], b_out.at[pl.ds(base, _RCH)],
                                  semw[buf]),
            pltpu.make_async_copy(nv_b[buf], n_out.at[pl.ds(base, _RCH)],
                                  semw[buf]),
        )

    rows_copy(0, 0).start()
    for cp in bn_copies(0, 0):
        cp.start()

    def chunk(i, buf):
        cc = 2 * i + buf

        @pl.when(cc >= 2)
        def _():
            for cp in out_copies(cc - 2, buf):
                cp.wait()

        nxt = cc + 1
        if buf == 0:
            rows_copy(nxt, 1).start()
            for cp in bn_copies(nxt, 1):
                cp.start()
        else:
            @pl.when(nxt < _NCH)
            def _():
                rows_copy(nxt, 0).start()
                for cp in bn_copies(nxt, 0):
                    cp.start()

        rows_copy(cc, buf).wait()
        rows_v = rows_b[buf]

        def tok(t, carry):
            tg = cc * _TCH + t
            xs = [in_v[tg, pl.ds(16 * j, 16)] for j in range(8)]
            for k in range(_K):
                r = t * _K + k
                acc = xs[0] * rows_v[r, pl.ds(0, 16)]
                for j in range(1, 8):
                    acc = acc + xs[j] * rows_v[r, pl.ds(16 * j, 16)]
                acc_v[pl.ds(r * 16, 16)] = acc
            return carry

        lax.fori_loop(0, _TCH, tok, 0, unroll=False)

        lanes = lax.iota(jnp.int32, 16)

        def grp(g, carry):
            rid = (g * 16 + lanes) * 16
            s = plsc.load_gather(acc_v, [rid])
            for j in range(1, 16):
                s = s + plsc.load_gather(acc_v, [rid + j])
            out_b[buf][pl.ds(g * 16, 16)] = s
            return carry

        lax.fori_loop(0, _RPAD // 16, grp, 0, unroll=False)

        for cp in bn_copies(cc, buf):
            cp.wait()
        for cp in out_copies(cc, buf):
            cp.start()

    def loop_body(i, carry):
        chunk(i, 0)
        chunk(i, 1)
        return carry

    lax.fori_loop(0, _NCH // 2, loop_body, 0, unroll=False)

    for buf in (0, 1):
        for cp in out_copies(_NCH - 2 + buf, buf):
            cp.wait()


_sc_gather_dot = functools.partial(
    pl.kernel,
    out_type=[jax.ShapeDtypeStruct((_NFLAT,), jnp.float32)] * 3,
    mesh=plsc.VectorSubcoreMesh(core_axis_name="c", subcore_axis_name="s"),
    compiler_params=pltpu.CompilerParams(needs_layout_passes=False),
    scratch_types=[
        pltpu.VMEM((_TPW, _E), jnp.float32),
        pltpu.VMEM((_NCH, _RCH), jnp.int32),
        pltpu.VMEM((_RCH, _E), jnp.float32),
        pltpu.VMEM((_RCH, _E), jnp.float32),
        pltpu.VMEM((_RPAD * 16,), jnp.float32),
        pltpu.VMEM((_RPAD,), jnp.float32),
        pltpu.VMEM((_RPAD,), jnp.float32),
        pltpu.VMEM((_RCH,), jnp.float32),
        pltpu.VMEM((_RCH,), jnp.float32),
        pltpu.VMEM((_RCH,), jnp.float32),
        pltpu.VMEM((_RCH,), jnp.float32),
        pltpu.SemaphoreType.DMA,
        pltpu.SemaphoreType.DMA,
        pltpu.SemaphoreType.DMA,
        pltpu.SemaphoreType.DMA,
        pltpu.SemaphoreType.DMA,
        pltpu.SemaphoreType.DMA,
        pltpu.SemaphoreType.DMA,
        pltpu.SemaphoreType.DMA,
    ],
)(_sc_body)


def _tc_loss_body(d_ref, bv_ref, nv_ref, o_ref):
    out = d_ref[...] + bv_ref[...]
    probs = jnp.exp(out - _NORM)
    c = _NR * nv_ref[...]
    rows = lax.broadcasted_iota(jnp.int32, d_ref.shape, 0)
    cols = lax.broadcasted_iota(jnp.int32, d_ref.shape, 1)
    k = (rows * d_ref.shape[1] + cols) % _K
    num = jnp.where(k == 0, probs, c)
    terms = jnp.log(num / (probs + c))
    o_ref[0, 0] = -jnp.sum(terms) / _N


def kernel(input, target, noise_samples, W, b, noise):
    idx = jnp.concatenate(
        [target[:, None].astype(jnp.int32),
         noise_samples.astype(jnp.int32)], axis=1)
    idx2d = idx.reshape(_NW * _NCH, _RCH)

    dot_flat, b_flat, n_flat = _sc_gather_dot(
        W, input, idx2d, b, noise)

    shp = (_NFLAT // _E, _E)
    loss = pl.pallas_call(
        _tc_loss_body,
        out_shape=jax.ShapeDtypeStruct((1, 1), jnp.float32),
        out_specs=pl.BlockSpec(memory_space=pltpu.SMEM),
    )(dot_flat.reshape(shp), b_flat.reshape(shp), n_flat.reshape(shp))
    return loss[0, 0]


# speedup vs baseline: 1.0610x; 1.0610x over previous
import functools

import jax
import jax.numpy as jnp
from jax import lax
from jax.experimental import pallas as pl
from jax.experimental.pallas import tpu as pltpu
from jax.experimental.pallas import tpu_sc as plsc

_N = 4096
_E = 128
_K = 26
_NORM = 9.0
_NR = 25.0

_NW = 32
_TPW = _N // _NW
_TCH = 4
_RCH = _TCH * _K
_NCH = _TPW // _TCH
_RPW = _TPW * _K
_RPAD = 112
_STR = 17
_NFLAT = _N * _K


def _sc_body(w_hbm, inp_hbm, idx_hbm, b_hbm, nz_hbm,
             dot_out, b_out, n_out,
             in_v, idx_v, rows0, rows1, acc_v,
             out0, out1, bv0, bv1, nv0, nv1,
             sem0, sem1, semb0, semb1, semn0, semn1, semw0, semw1):
    nc = 2
    wid = lax.axis_index("s") * nc + lax.axis_index("c")
    tok0 = wid * _TPW

    rows_b = (rows0, rows1)
    out_b = (out0, out1)
    bv_b = (bv0, bv1)
    nv_b = (nv0, nv1)
    semr = (sem0, sem1)
    semb = (semb0, semb1)
    semn = (semn0, semn1)
    semw = (semw0, semw1)

    pltpu.sync_copy(idx_hbm.at[pl.ds(wid * _NCH, _NCH)], idx_v)
    pltpu.sync_copy(inp_hbm.at[pl.ds(tok0, _TPW)], in_v)

    def rows_copy(cc, buf):
        return pltpu.make_async_copy(
            w_hbm.at[idx_v.at[cc]], rows_b[buf], semr[buf])

    def bn_copies(cc, buf):
        return (
            pltpu.make_async_copy(b_hbm.at[idx_v.at[cc]], bv_b[buf],
                                  semb[buf]),
            pltpu.make_async_copy(nz_hbm.at[idx_v.at[cc]], nv_b[buf],
                                  semn[buf]),
        )

    def out_copies(cc, buf):
        base = wid * _RPW + cc * _RCH
        return (
            pltpu.make_async_copy(out_b[buf].at[pl.ds(0, _RCH)],
                                  dot_out.at[pl.ds(base, _RCH)], semw[buf]),
            pltpu.make_async_copy(bv_b[buf], b_out.at[pl.ds(base, _RCH)],
                                  semw[buf]),
            pltpu.make_async_copy(nv_b[buf], n_out.at[pl.ds(base, _RCH)],
                                  semw[buf]),
        )

    rows_copy(0, 0).start()
    for cp in bn_copies(0, 0):
        cp.start()

    def chunk(i, buf):
        cc = 2 * i + buf

        @pl.when(cc >= 2)
        def _():
            for cp in out_copies(cc - 2, buf):
                cp.wait()

        nxt = cc + 1
        if buf == 0:
            rows_copy(nxt, 1).start()
            for cp in bn_copies(nxt, 1):
                cp.start()
        else:
            @pl.when(nxt < _NCH)
            def _():
                rows_copy(nxt, 0).start()
                for cp in bn_copies(nxt, 0):
                    cp.start()

        rows_copy(cc, buf).wait()
        rows_v = rows_b[buf]

        def tok(t, carry):
            tg = cc * _TCH + t
            xs = [in_v[tg, pl.ds(16 * j, 16)] for j in range(8)]
            for k in range(_K):
                r = t * _K + k
                acc = xs[0] * rows_v[r, pl.ds(0, 16)]
                for j in range(1, 8):
                    acc = acc + xs[j] * rows_v[r, pl.ds(16 * j, 16)]
                acc_v[pl.ds(r * _STR, 16)] = acc
            return carry

        lax.fori_loop(0, _TCH, tok, 0, unroll=False)

        lanes = lax.iota(jnp.int32, 16)

        def grp(g, carry):
            rid = (g * 16 + lanes) * _STR
            s = plsc.load_gather(acc_v, [rid])
            for j in range(1, 16):
                s = s + plsc.load_gather(acc_v, [rid + j])
            out_b[buf][pl.ds(g * 16, 16)] = s
            return carry

        lax.fori_loop(0, _RPAD // 16, grp, 0, unroll=False)

        for cp in bn_copies(cc, buf):
            cp.wait()
        for cp in out_copies(cc, buf):
            cp.start()

    def loop_body(i, carry):
        chunk(i, 0)
        chunk(i, 1)
        return carry

    lax.fori_loop(0, _NCH // 2, loop_body, 0, unroll=False)

    for buf in (0, 1):
        for cp in out_copies(_NCH - 2 + buf, buf):
            cp.wait()


_sc_gather_dot = functools.partial(
    pl.kernel,
    out_type=[jax.ShapeDtypeStruct((_NFLAT,), jnp.float32)] * 3,
    mesh=plsc.VectorSubcoreMesh(core_axis_name="c", subcore_axis_name="s"),
    compiler_params=pltpu.CompilerParams(needs_layout_passes=False),
    scratch_types=[
        pltpu.VMEM((_TPW, _E), jnp.float32),
        pltpu.VMEM((_NCH, _RCH), jnp.int32),
        pltpu.VMEM((_RCH, _E), jnp.float32),
        pltpu.VMEM((_RCH, _E), jnp.float32),
        pltpu.VMEM((_RPAD * _STR,), jnp.float32),
        pltpu.VMEM((_RPAD,), jnp.float32),
        pltpu.VMEM((_RPAD,), jnp.float32),
        pltpu.VMEM((_RCH,), jnp.float32),
        pltpu.VMEM((_RCH,), jnp.float32),
        pltpu.VMEM((_RCH,), jnp.float32),
        pltpu.VMEM((_RCH,), jnp.float32),
        pltpu.SemaphoreType.DMA,
        pltpu.SemaphoreType.DMA,
        pltpu.SemaphoreType.DMA,
        pltpu.SemaphoreType.DMA,
        pltpu.SemaphoreType.DMA,
        pltpu.SemaphoreType.DMA,
        pltpu.SemaphoreType.DMA,
        pltpu.SemaphoreType.DMA,
    ],
)(_sc_body)


def _tc_loss_body(d_ref, bv_ref, nv_ref, o_ref):
    out = d_ref[...] + bv_ref[...]
    probs = jnp.exp(out - _NORM)
    c = _NR * nv_ref[...]
    rows = lax.broadcasted_iota(jnp.int32, d_ref.shape, 0)
    cols = lax.broadcasted_iota(jnp.int32, d_ref.shape, 1)
    k = (rows * d_ref.shape[1] + cols) % _K
    num = jnp.where(k == 0, probs, c)
    terms = jnp.log(num / (probs + c))
    o_ref[0, 0] = -jnp.sum(terms) / _N


def kernel(input, target, noise_samples, W, b, noise):
    idx = jnp.concatenate(
        [target[:, None].astype(jnp.int32),
         noise_samples.astype(jnp.int32)], axis=1)
    idx2d = idx.reshape(_NW * _NCH, _RCH)

    dot_flat, b_flat, n_flat = _sc_gather_dot(
        W, input, idx2d, b, noise)

    shp = (_NFLAT // _E, _E)
    loss = pl.pallas_call(
        _tc_loss_body,
        out_shape=jax.ShapeDtypeStruct((1, 1), jnp.float32),
        out_specs=pl.BlockSpec(memory_space=pltpu.SMEM),
    )(dot_flat.reshape(shp), b_flat.reshape(shp), n_flat.reshape(shp))
    return loss[0, 0]
